# baseline (speedup 1.0000x reference)
import functools

import jax
import jax.numpy as jnp
from jax import lax
from jax.experimental import pallas as pl
from jax.experimental.pallas import tpu as pltpu
from jax.experimental.pallas import tpu_sc as plsc

_E = 64
_TOPK = 8
_BLK = 1024

_NC = 2
_NS = 16
_NW = _NC * _NS
_L = 16
_G = 2
_H = 256
_RSC = 64
_TSC = _NW * _RSC


def _router_block(x_ref, w_ref, scores_ref, wts_ref, idx_ref):
    logits = jax.lax.dot_general(
        x_ref[...], w_ref[...], (((1,), (1,)), ((), ())),
        preferred_element_type=jnp.float32)
    m = jnp.max(logits, axis=-1, keepdims=True)
    e = jnp.exp(logits - m)
    s = e / jnp.sum(e, axis=-1, keepdims=True)
    scores_ref[...] = s

    iota = jax.lax.broadcasted_iota(jnp.int32, s.shape, 1)
    work = s
    wcols, icols = [], []
    for _ in range(_TOPK):
        mk = jnp.max(work, axis=-1, keepdims=True)
        hit = work == mk
        ik = jnp.min(jnp.where(hit, iota, _E), axis=-1, keepdims=True)
        wcols.append(mk)
        icols.append(ik)
        work = jnp.where(iota == ik, -jnp.inf, work)
    wts_ref[...] = jnp.concatenate(wcols, axis=1)
    idx_ref[...] = jnp.concatenate(icols, axis=1)


def _tc_router(xf, W, t1):
    hs = xf.shape[1]
    return pl.pallas_call(
        _router_block,
        grid=(t1 // _BLK,),
        in_specs=[
            pl.BlockSpec((_BLK, hs), lambda i: (i, 0)),
            pl.BlockSpec((_E, hs), lambda i: (0, 0)),
        ],
        out_specs=[
            pl.BlockSpec((_BLK, _E), lambda i: (i, 0)),
            pl.BlockSpec((_BLK, _TOPK), lambda i: (i, 0)),
            pl.BlockSpec((_BLK, _TOPK), lambda i: (i, 0)),
        ],
        out_shape=[
            jax.ShapeDtypeStruct((t1, _E), jnp.float32),
            jax.ShapeDtypeStruct((t1, _TOPK), jnp.float32),
            jax.ShapeDtypeStruct((t1, _TOPK), jnp.int32),
        ],
    )(xf, W)


def _top8_sorted(svecs, lane):
    mk = mv = None
    for j in range(_E // _L):
        sk, sv = plsc.sort_key_val(svecs[j], lane + (j * _L), descending=True)
        if mk is None:
            mk, mv = sk, sv
        else:
            ck = jnp.where(lane < _TOPK, mk, lax.rev(sk, (0,)))
            cv = jnp.where(lane < _TOPK, mv, lax.rev(sv, (0,)))
            mk, mv = plsc.sort_key_val(ck, cv, descending=True)
    return mk, mv


def _round_to_bf16(v):
    u = plsc.bitcast(v, jnp.int32)
    r = u + jnp.int32(0x7FFF) + ((u >> jnp.int32(16)) & jnp.int32(1))
    r = r & jnp.int32(-65536)
    return plsc.bitcast(r, jnp.float32)


def _sc_router(xf, Wt, t1):
    hs = xf.shape[1]
    n_chunks = hs // _H
    n_groups = _RSC // _G

    mesh = plsc.VectorSubcoreMesh(core_axis_name="c", subcore_axis_name="s")

    @functools.partial(
        pl.kernel,
        out_type=[
            jax.ShapeDtypeStruct((_TSC, _E), jnp.float32),
            jax.ShapeDtypeStruct((_TSC, _L), jnp.float32),
            jax.ShapeDtypeStruct((_TSC, _L), jnp.int32),
        ],
        mesh=mesh,
        scratch_types=[
            pltpu.VMEM((_RSC, _H), jnp.float32),
            pltpu.VMEM((_H, 2 * _E), jnp.float32),
            pltpu.VMEM((_RSC, _E), jnp.float32),
            pltpu.VMEM((_RSC, _E), jnp.float32),
            pltpu.VMEM((_RSC, _L), jnp.float32),
            pltpu.VMEM((_RSC, _L), jnp.int32),
        ],
        compiler_params=pltpu.CompilerParams(needs_layout_passes=False, internal_scratch_in_bytes=65536),
    )
    def k(x_hbm, wt_hbm, sc_hbm, wo_hbm, io_hbm,
          xbuf, wbuf, acc, sbuf, obuf_w, obuf_i):
        wid = lax.axis_index("s") * _NC + lax.axis_index("c")
        row0 = t1 + wid * _RSC
        out0 = wid * _RSC
        lane = lax.iota(jnp.int32, _L)
        zero = jnp.zeros((_L,), jnp.float32)

        def zbody(g, carry):
            for rr in range(_G):
                for j in range(_E // _L):
                    acc[g * _G + rr, pl.ds(j * _L, _L)] = zero
            return carry

        lax.fori_loop(0, n_groups, zbody, 0)

        def chunk_body(c, carry):
            h0 = c * _H
            pltpu.sync_copy(x_hbm.at[pl.ds(row0, _RSC), pl.ds(h0, _H)], xbuf)
            pltpu.sync_copy(wt_hbm.at[pl.ds(h0, _H), :], wbuf)

            def round_body(i, carry2):
                for cc in range(_H // _L):
                    xbuf[i, pl.ds(cc * _L, _L)] = _round_to_bf16(
                        xbuf[i, pl.ds(cc * _L, _L)])
                return carry2

            lax.fori_loop(0, _RSC, round_body, 0)

            def group_body(g, carry2):
                accs = [
                    [acc[g * _G + rr, pl.ds(j * _L, _L)]
                     for j in range(_E // _L)]
                    for rr in range(_G)
                ]

                def h_body(h, accs_flat):
                    a = [list(accs_flat[rr * 4:(rr + 1) * 4])
                         for rr in range(_G)]
                    hc = (h // _L) * _L
                    l = jnp.full((_L, 1), h % _L, jnp.int32)
                    wv = [wbuf[h, pl.ds(j * _L, _L)]
                          for j in range(_E // _L)]
                    for rr in range(_G):
                        xv = xbuf[g * _G + rr, pl.ds(hc, _L)]
                        xb = lax.gather(
                            xv, l,
                            lax.GatherDimensionNumbers(
                                offset_dims=(),
                                collapsed_slice_dims=(0,),
                                start_index_map=(0,)),
                            (1,),
                            mode=lax.GatherScatterMode.PROMISE_IN_BOUNDS)
                        for j in range(_E // _L):
                            a[rr][j] = a[rr][j] + xb * wv[j]
                    return tuple(v for row in a for v in row)

                accs_flat = lax.fori_loop(
                    0, _H, h_body,
                    tuple(v for row in accs for v in row))
                for rr in range(_G):
                    for j in range(_E // _L):
                        acc[g * _G + rr, pl.ds(j * _L, _L)] = (
                            accs_flat[rr * 4 + j])
                return carry2

            lax.fori_loop(0, n_groups, group_body, 0)
            return carry

        lax.fori_loop(0, n_chunks, chunk_body, 0)

        def tail_body(g, carry):
            for rr in range(_G):
                r = g * _G + rr
                av = [acc[r, pl.ds(j * _L, _L)] for j in range(_E // _L)]
                m = jnp.max(jnp.maximum(jnp.maximum(av[0], av[1]),
                                        jnp.maximum(av[2], av[3])))
                ev = [jnp.exp(v - m) for v in av]
                tot = jnp.sum(ev[0] + ev[1] + ev[2] + ev[3])
                sv = [v / tot for v in ev]
                for j in range(_E // _L):
                    sbuf[r, pl.ds(j * _L, _L)] = sv[j]
                _, mv = _top8_sorted(av, lane)
                rowv = jnp.full((_L,), r, jnp.int32)
                obuf_w[r, :] = plsc.load_gather(sbuf, [rowv, mv])
                obuf_i[r, :] = mv
            return carry

        lax.fori_loop(0, n_groups, tail_body, 0)
        pltpu.sync_copy(sbuf, sc_hbm.at[pl.ds(out0, _RSC), :])
        pltpu.sync_copy(obuf_w, wo_hbm.at[pl.ds(out0, _RSC), :])
        pltpu.sync_copy(obuf_i, io_hbm.at[pl.ds(out0, _RSC), :])

    return k(xf, Wt)


def kernel(x, W):
    sl, bs, hs = x.shape
    t = sl * bs
    xf = x.reshape(t, hs)
    t1 = t - _TSC
    sc_tc, wt_tc, ix_tc = _tc_router(xf, W, t1)
    wt_bf = W.T.astype(jnp.bfloat16).astype(jnp.float32)
    wt_pad = jnp.pad(wt_bf, ((0, 0), (0, _E)))
    sc_sc, wt_sc, ix_sc = _sc_router(xf, wt_pad, t1)
    scores = jnp.concatenate([sc_tc, sc_sc], axis=0)
    wts = jnp.concatenate([wt_tc, wt_sc[:, :_TOPK]], axis=0)
    idx = jnp.concatenate([ix_tc, ix_sc[:, :_TOPK]], axis=0)
    return scores, wts, idx


# speedup vs baseline: 1.0001x; 1.0001x over previous
import functools

import jax
import jax.numpy as jnp
from jax import lax
from jax.experimental import pallas as pl
from jax.experimental.pallas import tpu as pltpu
from jax.experimental.pallas import tpu_sc as plsc

_E = 64
_TOPK = 8
_BLK = 1024

_NC = 2
_NS = 16
_NW = _NC * _NS
_L = 16
_G = 2
_H = 256
_RSC = 64
_TSC = _NW * _RSC


def _router_block(x_ref, w_ref, scores_ref, wts_ref, idx_ref):
    logits = jax.lax.dot_general(
        x_ref[...], w_ref[...], (((1,), (1,)), ((), ())),
        preferred_element_type=jnp.float32)
    m = jnp.max(logits, axis=-1, keepdims=True)
    e = jnp.exp(logits - m)
    s = e / jnp.sum(e, axis=-1, keepdims=True)
    scores_ref[...] = s

    iota = jax.lax.broadcasted_iota(jnp.int32, s.shape, 1)
    work = s
    wcols, icols = [], []
    for _ in range(_TOPK):
        mk = jnp.max(work, axis=-1, keepdims=True)
        hit = work == mk
        ik = jnp.min(jnp.where(hit, iota, _E), axis=-1, keepdims=True)
        wcols.append(mk)
        icols.append(ik)
        work = jnp.where(iota == ik, -jnp.inf, work)
    wts_ref[...] = jnp.concatenate(wcols, axis=1)
    idx_ref[...] = jnp.concatenate(icols, axis=1)


def _tc_router(xf, W, t1):
    hs = xf.shape[1]
    return pl.pallas_call(
        _router_block,
        grid=(t1 // _BLK,),
        in_specs=[
            pl.BlockSpec((_BLK, hs), lambda i: (i, 0)),
            pl.BlockSpec((_E, hs), lambda i: (0, 0)),
        ],
        out_specs=[
            pl.BlockSpec((_BLK, _E), lambda i: (i, 0)),
            pl.BlockSpec((_BLK, _TOPK), lambda i: (i, 0)),
            pl.BlockSpec((_BLK, _TOPK), lambda i: (i, 0)),
        ],
        out_shape=[
            jax.ShapeDtypeStruct((t1, _E), jnp.float32),
            jax.ShapeDtypeStruct((t1, _TOPK), jnp.float32),
            jax.ShapeDtypeStruct((t1, _TOPK), jnp.int32),
        ],
    )(xf, W)


def _top8_sorted(svecs, lane):
    mk = mv = None
    for j in range(_E // _L):
        sk, sv = plsc.sort_key_val(svecs[j], lane + (j * _L), descending=True)
        if mk is None:
            mk, mv = sk, sv
        else:
            ck = jnp.where(lane < _TOPK, mk, lax.rev(sk, (0,)))
            cv = jnp.where(lane < _TOPK, mv, lax.rev(sv, (0,)))
            mk, mv = plsc.sort_key_val(ck, cv, descending=True)
    return mk, mv


def _round_to_bf16(v):
    u = plsc.bitcast(v, jnp.int32)
    r = u + jnp.int32(0x7FFF) + ((u >> jnp.int32(16)) & jnp.int32(1))
    r = r & jnp.int32(-65536)
    return plsc.bitcast(r, jnp.float32)


def _sc_router(xf, Wt, t1):
    hs = xf.shape[1]
    n_chunks = hs // _H
    n_groups = _RSC // _G

    mesh = plsc.VectorSubcoreMesh(core_axis_name="c", subcore_axis_name="s")

    @functools.partial(
        pl.kernel,
        out_type=[
            jax.ShapeDtypeStruct((_TSC, _E), jnp.float32),
            jax.ShapeDtypeStruct((_TSC, _L), jnp.float32),
            jax.ShapeDtypeStruct((_TSC, _L), jnp.int32),
        ],
        mesh=mesh,
        scratch_types=[
            pltpu.VMEM((_RSC, _H), jnp.float32),
            pltpu.VMEM((_H, 2 * _E), jnp.float32),
            pltpu.VMEM((_RSC, _E), jnp.float32),
            pltpu.VMEM((_RSC, _E), jnp.float32),
            pltpu.VMEM((_RSC, _L), jnp.float32),
            pltpu.VMEM((_RSC, _L), jnp.int32),
        ],
        compiler_params=pltpu.CompilerParams(needs_layout_passes=False, internal_scratch_in_bytes=65536),
    )
    def k(x_hbm, wt_hbm, sc_hbm, wo_hbm, io_hbm,
          xbuf, wbuf, acc, sbuf, obuf_w, obuf_i):
        wid = lax.axis_index("s") * _NC + lax.axis_index("c")
        row0 = t1 + wid * _RSC
        out0 = wid * _RSC
        lane = lax.iota(jnp.int32, _L)
        zero = jnp.zeros((_L,), jnp.float32)

        def zbody(g, carry):
            for rr in range(_G):
                for j in range(_E // _L):
                    acc[g * _G + rr, pl.ds(j * _L, _L)] = zero
            return carry

        lax.fori_loop(0, n_groups, zbody, 0)

        def chunk_body(c, carry):
            h0 = c * _H
            pltpu.sync_copy(x_hbm.at[pl.ds(row0, _RSC), pl.ds(h0, _H)], xbuf)
            pltpu.sync_copy(wt_hbm.at[pl.ds(h0, _H), :], wbuf)

            def round_body(i, carry2):
                for cc in range(_H // _L):
                    xbuf[i, pl.ds(cc * _L, _L)] = _round_to_bf16(
                        xbuf[i, pl.ds(cc * _L, _L)])
                return carry2

            lax.fori_loop(0, _RSC, round_body, 0)

            def group_body(g, carry2):
                accs = [
                    [acc[g * _G + rr, pl.ds(j * _L, _L)]
                     for j in range(_E // _L)]
                    for rr in range(_G)
                ]

                def h_body(h, accs_flat):
                    a = [list(accs_flat[rr * 4:(rr + 1) * 4])
                         for rr in range(_G)]
                    hc = (h // _L) * _L
                    l = jnp.full((_L, 1), h % _L, jnp.int32)
                    wv = [wbuf[h, pl.ds(j * _L, _L)]
                          for j in range(_E // _L)]
                    for rr in range(_G):
                        xv = xbuf[g * _G + rr, pl.ds(hc, _L)]
                        xb = lax.gather(
                            xv, l,
                            lax.GatherDimensionNumbers(
                                offset_dims=(),
                                collapsed_slice_dims=(0,),
                                start_index_map=(0,)),
                            (1,),
                            mode=lax.GatherScatterMode.PROMISE_IN_BOUNDS)
                        for j in range(_E // _L):
                            a[rr][j] = a[rr][j] + xb * wv[j]
                    return tuple(v for row in a for v in row)

                accs_flat = lax.fori_loop(
                    0, _H, h_body,
                    tuple(v for row in accs for v in row))
                for rr in range(_G):
                    for j in range(_E // _L):
                        acc[g * _G + rr, pl.ds(j * _L, _L)] = (
                            accs_flat[rr * 4 + j])
                return carry2

            lax.fori_loop(0, n_groups, group_body, 0)
            return carry

        lax.fori_loop(0, n_chunks, chunk_body, 0)

        def tail_body(g, carry):
            for rr in range(_G):
                r = g * _G + rr
                av = [acc[r, pl.ds(j * _L, _L)] for j in range(_E // _L)]
                m = jnp.max(jnp.maximum(jnp.maximum(av[0], av[1]),
                                        jnp.maximum(av[2], av[3])))
                ev = [jnp.exp(v - m) for v in av]
                tot = jnp.sum(ev[0] + ev[1] + ev[2] + ev[3])
                sv = [v / tot for v in ev]
                for j in range(_E // _L):
                    sbuf[r, pl.ds(j * _L, _L)] = sv[j]
                _, mv = _top8_sorted(av, lane)
                rowv = jnp.full((_L,), r, jnp.int32)
                obuf_w[r, :] = plsc.load_gather(sbuf, [rowv, mv])
                obuf_i[r, :] = mv
            return carry

        lax.fori_loop(0, n_groups, tail_body, 0)
        pltpu.sync_copy(sbuf, sc_hbm.at[pl.ds(out0, _RSC), :])
        pltpu.sync_copy(obuf_w, wo_hbm.at[pl.ds(out0, _RSC), :])
        pltpu.sync_copy(obuf_i, io_hbm.at[pl.ds(out0, _RSC), :])

    return k(xf, Wt)


def kernel(x, W):
    sl, bs, hs = x.shape
    t = sl * bs
    xf = x.reshape(t, hs)
    t1 = t - _TSC
    wt_bf = W.T.astype(jnp.bfloat16).astype(jnp.float32)
    wt_pad = jnp.pad(wt_bf, ((0, 0), (0, _E)))
    sc_sc, wt_sc, ix_sc = _sc_router(xf, wt_pad, t1)
    sc_tc, wt_tc, ix_tc = _tc_router(xf, W, t1)
    scores = jnp.concatenate([sc_tc, sc_sc], axis=0)
    wts = jnp.concatenate([wt_tc, wt_sc[:, :_TOPK]], axis=0)
    idx = jnp.concatenate([ix_tc, ix_sc[:, :_TOPK]], axis=0)
    return scores, wts, idx


# speedup vs baseline: 2.0825x; 2.0823x over previous
import jax
import jax.numpy as jnp
from jax.experimental import pallas as pl
from jax.experimental.pallas import tpu as pltpu

_E = 64
_TOPK = 8
_BLK = 2048
_KC = 1024


def _router_block(x_ref, w_ref, scores_ref, wts_ref, idx_ref, acc_ref):
    nk = pl.num_programs(1)
    k = pl.program_id(1)
    partial = jax.lax.dot_general(
        x_ref[...], w_ref[...], (((1,), (1,)), ((), ())),
        preferred_element_type=jnp.float32)

    @pl.when(k == 0)
    def _init():
        acc_ref[...] = partial

    @pl.when(k > 0)
    def _accum():
        acc_ref[...] += partial

    @pl.when(k == nk - 1)
    def _finish():
        logits = acc_ref[...]
        m = jnp.max(logits, axis=-1, keepdims=True)
        e = jnp.exp(logits - m)
        s = e / jnp.sum(e, axis=-1, keepdims=True)
        scores_ref[...] = s

        iota = jax.lax.broadcasted_iota(jnp.int32, s.shape, 1)
        work = s
        wcols, icols = [], []
        for _ in range(_TOPK):
            mk = jnp.max(work, axis=-1, keepdims=True)
            hit = work == mk
            ik = jnp.min(jnp.where(hit, iota, _E), axis=-1, keepdims=True)
            wcols.append(mk)
            icols.append(ik)
            work = jnp.where(iota == ik, -jnp.inf, work)
        wts_ref[...] = jnp.concatenate(wcols, axis=1)
        idx_ref[...] = jnp.concatenate(icols, axis=1)


def kernel(x, W):
    sl, bs, hs = x.shape
    t = sl * bs
    xf = x.reshape(t, hs)
    grid = (t // _BLK, hs // _KC)
    scores, wts, idx = pl.pallas_call(
        _router_block,
        grid=grid,
        in_specs=[
            pl.BlockSpec((_BLK, _KC), lambda i, k: (i, k)),
            pl.BlockSpec((_E, _KC), lambda i, k: (0, k)),
        ],
        out_specs=[
            pl.BlockSpec((_BLK, _E), lambda i, k: (i, 0)),
            pl.BlockSpec((_BLK, _TOPK), lambda i, k: (i, 0)),
            pl.BlockSpec((_BLK, _TOPK), lambda i, k: (i, 0)),
        ],
        out_shape=[
            jax.ShapeDtypeStruct((t, _E), jnp.float32),
            jax.ShapeDtypeStruct((t, _TOPK), jnp.float32),
            jax.ShapeDtypeStruct((t, _TOPK), jnp.int32),
        ],
        scratch_shapes=[pltpu.VMEM((_BLK, _E), jnp.float32)],
        compiler_params=pltpu.CompilerParams(
            dimension_semantics=("parallel", "arbitrary")),
    )(xf, W)
    return scores, wts, idx


# speedup vs baseline: 2.3041x; 1.1064x over previous
import jax
import jax.numpy as jnp
from jax.experimental import pallas as pl

_E = 64
_TOPK = 8
_BLK = 1024


def _router_block(x_ref, w_ref, scores_ref, wts_ref, idx_ref):
    logits = jax.lax.dot_general(
        x_ref[...], w_ref[...], (((1,), (1,)), ((), ())),
        preferred_element_type=jnp.float32)
    m = jnp.max(logits, axis=-1, keepdims=True)
    e = jnp.exp(logits - m)
    s = e / jnp.sum(e, axis=-1, keepdims=True)
    scores_ref[...] = s

    iota = jax.lax.broadcasted_iota(jnp.int32, s.shape, 1)
    work = s
    wcols, icols = [], []
    for _ in range(_TOPK):
        mk = jnp.max(work, axis=-1, keepdims=True)
        hit = work == mk
        ik = jnp.min(jnp.where(hit, iota, _E), axis=-1, keepdims=True)
        wcols.append(mk)
        icols.append(ik)
        work = jnp.where(iota == ik, -jnp.inf, work)
    wts_ref[...] = jnp.concatenate(wcols, axis=1)
    idx_ref[...] = jnp.concatenate(icols, axis=1)


def kernel(x, W):
    sl, bs, hs = x.shape
    t = sl * bs
    xf = x.reshape(t, hs)
    grid = (t // _BLK,)
    scores, wts, idx = pl.pallas_call(
        _router_block,
        grid=grid,
        in_specs=[
            pl.BlockSpec((_BLK, hs), lambda i: (i, 0)),
            pl.BlockSpec((_E, hs), lambda i: (0, 0)),
        ],
        out_specs=[
            pl.BlockSpec((_BLK, _E), lambda i: (i, 0)),
            pl.BlockSpec((_BLK, _TOPK), lambda i: (i, 0)),
            pl.BlockSpec((_BLK, _TOPK), lambda i: (i, 0)),
        ],
        out_shape=[
            jax.ShapeDtypeStruct((t, _E), jnp.float32),
            jax.ShapeDtypeStruct((t, _TOPK), jnp.float32),
            jax.ShapeDtypeStruct((t, _TOPK), jnp.int32),
        ],
    )(xf, W)
    return scores, wts, idx
